# baseline (speedup 1.0000x reference)
import functools

import jax
import jax.numpy as jnp
from jax import lax
from jax.experimental import pallas as pl
from jax.experimental.pallas import tpu as pltpu
from jax.experimental.pallas import tpu_sc as plsc


def _make_sc_kernel(q, k, d):
    info = plsc.get_sparse_core_info()
    nw = info.num_cores * info.num_subcores
    rows_per_w = q // nw
    mesh = plsc.VectorSubcoreMesh(core_axis_name="c", subcore_axis_name="s")
    nvec = k // 16

    @functools.partial(
        pl.kernel,
        mesh=mesh,
        out_type=jax.ShapeDtypeStruct((q, d, k), jnp.float32),
        scratch_types=[
            pltpu.VMEM((rows_per_w * d * 16,), jnp.float32),
            pltpu.VMEM((d, k), jnp.float32),
            pltpu.VMEM((d, k), jnp.float32),
            pltpu.VMEM((d, k), jnp.float32),
            pltpu.SemaphoreType.DMA,
            pltpu.SemaphoreType.DMA,
        ],
    )
    def sck(xs_hbm, yt_hbm, out_hbm, xs_v, yt_v, buf_a, buf_b, sem_a, sem_b):
        wid = lax.axis_index("s") * info.num_cores + lax.axis_index("c")
        base = wid * rows_per_w
        pltpu.sync_copy(xs_hbm.at[pl.ds(base * d * 16, rows_per_w * d * 16)], xs_v)
        pltpu.sync_copy(yt_hbm, yt_v)

        def compute_plane(r, buf):
            for dd in range(d):
                xs = xs_v[pl.ds((r * d + dd) * 16, 16)]

                def t_body(t, _, dd=dd, xs=xs):
                    for u in range(8):
                        sl = pl.ds((t * 8 + u) * 16, 16)
                        buf[dd, sl] = yt_v[dd, sl] + xs
                    return 0

                lax.fori_loop(0, nvec // 8, t_body, 0)

        def pair(p, _):
            r0 = 2 * p
            i0 = base + r0

            @pl.when(p > 0)
            def _():
                pltpu.make_async_copy(buf_a, out_hbm.at[i0 - 2], sem_a).wait()
                pltpu.make_async_copy(buf_b, out_hbm.at[i0 - 1], sem_b).wait()

            compute_plane(r0, buf_a)
            pltpu.async_copy(buf_a, out_hbm.at[i0], sem_a)
            compute_plane(r0 + 1, buf_b)
            pltpu.async_copy(buf_b, out_hbm.at[i0 + 1], sem_b)
            return 0

        lax.fori_loop(0, rows_per_w // 2, pair, 0)
        last = base + rows_per_w
        pltpu.make_async_copy(buf_a, out_hbm.at[last - 2], sem_a).wait()
        pltpu.make_async_copy(buf_b, out_hbm.at[last - 1], sem_b).wait()

    return sck


def kernel(query_size, key_size, x_emb, y_emb):
    q, d = x_emb.shape
    k, _ = y_emb.shape
    x_eff = jnp.take(x_emb, jnp.arange(q) + (query_size - q), axis=0)
    y_eff = jnp.take(y_emb, jnp.arange(k) + (key_size - k), axis=0)
    x_splat = jnp.repeat(x_eff.reshape(q * d), 16).reshape(q * d * 16)
    out3 = _make_sc_kernel(q, k, d)(x_splat, y_eff.T)
    return jnp.transpose(out3, (0, 2, 1))


# speedup vs baseline: 1.6248x; 1.6248x over previous
import jax
import jax.numpy as jnp
from jax.experimental import pallas as pl


def _outer_sum_kernel(x_ref, yt_ref, o_ref):
    o_ref[...] = x_ref[...][:, :, None] + yt_ref[...][None, :, :]


def kernel(query_size, key_size, x_emb, y_emb):
    q, d = x_emb.shape
    k, _ = y_emb.shape
    x_eff = jnp.take(x_emb, jnp.arange(q) + (query_size - q), axis=0)
    y_eff = jnp.take(y_emb, jnp.arange(k) + (key_size - k), axis=0)

    yt = y_eff.T
    bq = 32
    out3 = pl.pallas_call(
        _outer_sum_kernel,
        grid=(q // bq,),
        in_specs=[
            pl.BlockSpec((bq, d), lambda i: (i, 0)),
            pl.BlockSpec((d, k), lambda i: (0, 0)),
        ],
        out_specs=pl.BlockSpec((bq, d, k), lambda i: (i, 0, 0)),
        out_shape=jax.ShapeDtypeStruct((q, d, k), x_emb.dtype),
    )(x_eff, yt)
    return jnp.transpose(out3, (0, 2, 1))
